# baseline (speedup 1.0000x reference)
import jax
import jax.numpy as jnp
from jax.experimental import pallas as pl

_ROWS = 16384
_COLS = 128
_BLK = 2048


def _body(x_ref, o_ref):
    blk = x_ref[...]
    o_ref[...] = blk
    b = blk[:, 124:125]
    cvb = blk[:, 125:126]
    cvl = blk[:, 126:127]
    qg = blk[:, 127:128]
    c = (b * cvb) / (1.0 - cvb)
    d = c * ((1.0 - cvl) / cvl)
    e2 = qg * d
    o_ref[:, 125:126] = c
    o_ref[:, 126:127] = d - e2
    o_ref[:, 127:128] = e2


def kernel(x):
    grid = (_ROWS // _BLK,)
    return pl.pallas_call(
        _body,
        grid=grid,
        in_specs=[pl.BlockSpec((_BLK, _COLS), lambda i: (i, 0))],
        out_specs=pl.BlockSpec((_BLK, _COLS), lambda i: (i, 0)),
        out_shape=jax.ShapeDtypeStruct((_ROWS, _COLS), jnp.float32),
    )(x)


# speedup vs baseline: 1.2298x; 1.2298x over previous
import jax
import jax.numpy as jnp
from jax.experimental import pallas as pl

_ROWS = 16384
_COLS = 128
_BLK = 4096


def _body(x_ref, o_ref):
    blk = x_ref[...]
    o_ref[...] = blk
    b = blk[:, 124:125]
    cvb = blk[:, 125:126]
    cvl = blk[:, 126:127]
    qg = blk[:, 127:128]
    c = b / (1.0 / cvb - 1.0)
    d = c / cvl - c
    o_ref[:, 125:126] = c
    o_ref[:, 126:127] = (1.0 - qg) * d
    o_ref[:, 127:128] = qg * d


def kernel(x):
    grid = (_ROWS // _BLK,)
    return pl.pallas_call(
        _body,
        grid=grid,
        in_specs=[pl.BlockSpec((_BLK, _COLS), lambda i: (i, 0))],
        out_specs=pl.BlockSpec((_BLK, _COLS), lambda i: (i, 0)),
        out_shape=jax.ShapeDtypeStruct((_ROWS, _COLS), jnp.float32),
    )(x)
